# baseline (speedup 1.0000x reference)
import functools

import jax
import jax.numpy as jnp
from jax import lax
from jax.experimental import pallas as pl
from jax.experimental.pallas import tpu as pltpu
from jax.experimental.pallas import tpu_sc as plsc

B = 16384
L = 50
D = 32
NW = 32
EPW = B // NW
CB = 32
NCH = EPW // CB
ROWS = CB * L
HALF = D // 2

V = 1000000
BLK = 128
NFULL = V // BLK
TAIL = V - NFULL * BLK
NBLK = NFULL + 1
VPAD = NBLK * BLK
KMAX = (NBLK + NW - 1) // NW


def _detranspose(table_t):
    mesh = plsc.VectorSubcoreMesh(core_axis_name="c", subcore_axis_name="s")

    @functools.partial(
        pl.kernel,
        mesh=mesh,
        out_type=jax.ShapeDtypeStruct((VPAD * D,), jnp.float32),
        scratch_types=[
            pltpu.VMEM((8, BLK), jnp.float32),
            pltpu.VMEM((8, BLK), jnp.float32),
            pltpu.VMEM((8, BLK), jnp.float32),
            pltpu.VMEM((8, BLK), jnp.float32),
            pltpu.VMEM((8, BLK), jnp.float32),
            pltpu.VMEM((8, BLK), jnp.float32),
            pltpu.VMEM((8, BLK), jnp.float32),
            pltpu.VMEM((8, BLK), jnp.float32),
            pltpu.VMEM((BLK * D,), jnp.float32),
            pltpu.VMEM((BLK * D,), jnp.float32),
            pltpu.SemaphoreType.DMA,
            pltpu.SemaphoreType.DMA,
            pltpu.SemaphoreType.DMA,
            pltpu.SemaphoreType.DMA,
        ],
        compiler_params=pltpu.CompilerParams(
            use_tc_tiling_on_sc=True, needs_layout_passes=False
        ),
    )
    def detr(t_hbm, o_hbm, b00, b01, b02, b03, b10, b11, b12, b13,
             st0, st1, gs0, gs1, os0, os1):
        wid = lax.axis_index("s") * 2 + lax.axis_index("c")
        blks = ((b00, b01, b02, b03), (b10, b11, b12, b13))
        sts = (st0, st1)
        gsems = (gs0, gs1)
        osems = (os0, os1)

        io = lax.iota(jnp.int32, 16)
        c8 = io & 7
        hlf = io >> 3
        sbase = [hlf * D + g * 8 + c8 for g in range(4)]

        def fire(k, p):
            tc = wid + k * NW

            @pl.when(tc < NFULL)
            def _():
                for g in range(4):
                    pltpu.async_copy(
                        t_hbm.at[pl.ds(g * 8, 8), pl.ds(tc * BLK, BLK)],
                        blks[p][g],
                        gsems[p],
                    )

            @pl.when(tc == NFULL)
            def _():
                for g in range(4):
                    pltpu.async_copy(
                        t_hbm.at[pl.ds(g * 8, 8), pl.ds(tc * BLK, TAIL)],
                        blks[p][g].at[:, pl.ds(0, TAIL)],
                        gsems[p],
                    )

        def transpose_rows(p, nrow):
            def row_body(i2, carry):
                i = i2 * 2
                col = hlf + i
                off = i * D
                for g in range(4):
                    vals = plsc.load_gather(blks[p][g], [c8, col])
                    plsc.store_scatter(sts[p], [sbase[g] + off], vals)
                return carry

            lax.fori_loop(0, nrow // 2, row_body, 0)

        def compute(k, p):
            tc = wid + k * NW

            @pl.when(tc <= NFULL)
            def _():
                pltpu.make_async_copy(
                    sts[p], o_hbm.at[pl.ds(0, BLK * D)], osems[p]
                ).wait()

            @pl.when(tc < NFULL)
            def _():
                for g in range(4):
                    pltpu.make_async_copy(
                        t_hbm.at[pl.ds(0, 8), pl.ds(0, BLK)],
                        blks[p][g],
                        gsems[p],
                    ).wait()
                transpose_rows(p, BLK)

            @pl.when(tc == NFULL)
            def _():
                for g in range(4):
                    pltpu.make_async_copy(
                        t_hbm.at[pl.ds(0, 8), pl.ds(0, TAIL)],
                        blks[p][g].at[:, pl.ds(0, TAIL)],
                        gsems[p],
                    ).wait()
                transpose_rows(p, TAIL)

            @pl.when(tc <= NFULL)
            def _():
                pltpu.async_copy(
                    sts[p],
                    o_hbm.at[pl.ds(tc * (BLK * D), BLK * D)],
                    osems[p],
                )

        for p in range(2):
            pltpu.async_copy(
                sts[p],
                o_hbm.at[pl.ds((wid + p * NW) * (BLK * D), BLK * D)],
                osems[p],
            )

        fire(0, 0)

        def loop_body(k2, carry):
            for ph in range(2):
                k = k2 * 2 + ph
                p = ph
                fire(k + 1, 1 - p)
                compute(k, p)
            return carry

        lax.fori_loop(0, KMAX // 2, loop_body, 0)
        compute(KMAX - 1, 0)

        for p in range(2):
            pltpu.make_async_copy(
                sts[p], o_hbm.at[pl.ds(0, BLK * D)], osems[p]
            ).wait()

    return detr(table_t)


def _bow(batch, table_lin):
    mesh = plsc.VectorSubcoreMesh(core_axis_name="c", subcore_axis_name="s")

    @functools.partial(
        pl.kernel,
        mesh=mesh,
        out_type=jax.ShapeDtypeStruct((B * D,), jnp.float32),
        scratch_types=[
            pltpu.VMEM((CB, L), jnp.int32),
            pltpu.VMEM((ROWS, D), jnp.float32),
            pltpu.VMEM((CB * D,), jnp.float32),
            pltpu.SemaphoreType.DMA,
        ],
        compiler_params=pltpu.CompilerParams(use_tc_tiling_on_sc=False),
    )
    def bow(idx_hbm, table_hbm, out_hbm, idx_v, rows_v, out_v, gsem):
        wid = lax.axis_index("s") * 2 + lax.axis_index("c")

        def chunk_body(c, carry):
            b0 = wid * EPW + c * CB
            pltpu.sync_copy(idx_hbm.at[pl.ds(b0, CB)], idx_v)
            gcopies = [
                pltpu.async_copy(
                    table_hbm.at[idx_v.at[j]],
                    rows_v.at[pl.ds(j * L, L)],
                    gsem,
                )
                for j in range(CB)
            ]
            for cp in gcopies:
                cp.wait()

            def elem_body(e, carry2):
                r0 = e * L
                a0 = rows_v[r0, 0:HALF] + rows_v[r0 + 1, 0:HALF]
                b0_ = rows_v[r0, HALF:D] + rows_v[r0 + 1, HALF:D]
                a1 = rows_v[r0 + 2, 0:HALF] + rows_v[r0 + 3, 0:HALF]
                b1 = rows_v[r0 + 2, HALF:D] + rows_v[r0 + 3, HALF:D]
                for l in range(4, L, 2):
                    a0 = a0 + rows_v[r0 + l, 0:HALF]
                    b0_ = b0_ + rows_v[r0 + l, HALF:D]
                    a1 = a1 + rows_v[r0 + l + 1, 0:HALF]
                    b1 = b1 + rows_v[r0 + l + 1, HALF:D]
                o0 = e * D
                out_v[pl.ds(o0, HALF)] = (a0 + a1) * (1.0 / L)
                out_v[pl.ds(o0 + HALF, HALF)] = (b0_ + b1) * (1.0 / L)
                return carry2

            lax.fori_loop(0, CB, elem_body, 0)
            pltpu.sync_copy(out_v, out_hbm.at[pl.ds(b0 * D, CB * D)])
            return carry

        lax.fori_loop(0, NCH, chunk_body, 0)

    return bow(batch, table_lin)


def kernel(batch, table):
    tlin = _detranspose(table.T)
    table_lin = tlin[: V * D].reshape(V, D)
    return _bow(batch, table_lin).reshape(B, D)


# speedup vs baseline: 1.1296x; 1.1296x over previous
import functools

import jax
import jax.numpy as jnp
from jax import lax
from jax.experimental import pallas as pl
from jax.experimental.pallas import tpu as pltpu
from jax.experimental.pallas import tpu_sc as plsc

B = 16384
L = 50
D = 32
NW = 32
EPW = B // NW
CB = 32
NCH = EPW // CB
ROWS = CB * L
HALF = D // 2

V = 1000000
BLK = 128
NFULL = V // BLK
TAIL = V - NFULL * BLK
NBLK = NFULL + 1
VPAD = NBLK * BLK
KMAX = (NBLK + NW - 1) // NW


def _detranspose(table_t):
    mesh = plsc.VectorSubcoreMesh(core_axis_name="c", subcore_axis_name="s")

    @functools.partial(
        pl.kernel,
        mesh=mesh,
        out_type=jax.ShapeDtypeStruct((VPAD * D,), jnp.float32),
        scratch_types=[
            pltpu.VMEM((8, BLK), jnp.float32),
            pltpu.VMEM((8, BLK), jnp.float32),
            pltpu.VMEM((8, BLK), jnp.float32),
            pltpu.VMEM((8, BLK), jnp.float32),
            pltpu.VMEM((8, BLK), jnp.float32),
            pltpu.VMEM((8, BLK), jnp.float32),
            pltpu.VMEM((8, BLK), jnp.float32),
            pltpu.VMEM((8, BLK), jnp.float32),
            pltpu.VMEM((BLK * D,), jnp.float32),
            pltpu.VMEM((BLK * D,), jnp.float32),
            pltpu.SemaphoreType.DMA,
            pltpu.SemaphoreType.DMA,
            pltpu.SemaphoreType.DMA,
            pltpu.SemaphoreType.DMA,
        ],
        compiler_params=pltpu.CompilerParams(
            use_tc_tiling_on_sc=True, needs_layout_passes=False
        ),
    )
    def detr(t_hbm, o_hbm, b00, b01, b02, b03, b10, b11, b12, b13,
             st0, st1, gs0, gs1, os0, os1):
        wid = lax.axis_index("s") * 2 + lax.axis_index("c")
        blks = ((b00, b01, b02, b03), (b10, b11, b12, b13))
        sts = (st0, st1)
        gsems = (gs0, gs1)
        osems = (os0, os1)

        io = lax.iota(jnp.int32, 16)
        c8 = io & 7
        hlf = io >> 3
        sbase = [hlf * D + g * 8 + c8 for g in range(4)]

        def fire(k, p):
            tc = wid + k * NW

            @pl.when(tc < NFULL)
            def _():
                for g in range(4):
                    pltpu.async_copy(
                        t_hbm.at[pl.ds(g * 8, 8), pl.ds(tc * BLK, BLK)],
                        blks[p][g],
                        gsems[p],
                    )

            @pl.when(tc == NFULL)
            def _():
                for g in range(4):
                    pltpu.async_copy(
                        t_hbm.at[pl.ds(g * 8, 8), pl.ds(tc * BLK, TAIL)],
                        blks[p][g].at[:, pl.ds(0, TAIL)],
                        gsems[p],
                    )

        def transpose_rows(p, nrow):
            unroll = 8

            def row_body(j, carry):
                i0 = j * (2 * unroll)
                for u in range(unroll):
                    i = i0 + 2 * u
                    col = hlf + i
                    off = i * D
                    for g in range(4):
                        vals = plsc.load_gather(blks[p][g], [c8, col])
                        plsc.store_scatter(sts[p], [sbase[g] + off], vals)
                return carry

            lax.fori_loop(0, nrow // (2 * unroll), row_body, 0)

        def compute(k, p):
            tc = wid + k * NW

            @pl.when(tc <= NFULL)
            def _():
                pltpu.make_async_copy(
                    sts[p], o_hbm.at[pl.ds(0, BLK * D)], osems[p]
                ).wait()

            @pl.when(tc < NFULL)
            def _():
                for g in range(4):
                    pltpu.make_async_copy(
                        t_hbm.at[pl.ds(0, 8), pl.ds(0, BLK)],
                        blks[p][g],
                        gsems[p],
                    ).wait()
                transpose_rows(p, BLK)

            @pl.when(tc == NFULL)
            def _():
                for g in range(4):
                    pltpu.make_async_copy(
                        t_hbm.at[pl.ds(0, 8), pl.ds(0, TAIL)],
                        blks[p][g].at[:, pl.ds(0, TAIL)],
                        gsems[p],
                    ).wait()
                transpose_rows(p, TAIL)

            @pl.when(tc <= NFULL)
            def _():
                pltpu.async_copy(
                    sts[p],
                    o_hbm.at[pl.ds(tc * (BLK * D), BLK * D)],
                    osems[p],
                )

        for p in range(2):
            pltpu.async_copy(
                sts[p],
                o_hbm.at[pl.ds((wid + p * NW) * (BLK * D), BLK * D)],
                osems[p],
            )

        fire(0, 0)

        def loop_body(k2, carry):
            for ph in range(2):
                k = k2 * 2 + ph
                p = ph
                fire(k + 1, 1 - p)
                compute(k, p)
            return carry

        lax.fori_loop(0, KMAX // 2, loop_body, 0)
        compute(KMAX - 1, 0)

        for p in range(2):
            pltpu.make_async_copy(
                sts[p], o_hbm.at[pl.ds(0, BLK * D)], osems[p]
            ).wait()

    return detr(table_t)


def _bow(batch, table_lin):
    mesh = plsc.VectorSubcoreMesh(core_axis_name="c", subcore_axis_name="s")

    @functools.partial(
        pl.kernel,
        mesh=mesh,
        out_type=jax.ShapeDtypeStruct((B * D,), jnp.float32),
        scratch_types=[
            pltpu.VMEM((CB, L), jnp.int32),
            pltpu.VMEM((ROWS, D), jnp.float32),
            pltpu.VMEM((CB * D,), jnp.float32),
            pltpu.SemaphoreType.DMA,
        ],
        compiler_params=pltpu.CompilerParams(use_tc_tiling_on_sc=False),
    )
    def bow(idx_hbm, table_hbm, out_hbm, idx_v, rows_v, out_v, gsem):
        wid = lax.axis_index("s") * 2 + lax.axis_index("c")

        def chunk_body(c, carry):
            b0 = wid * EPW + c * CB
            pltpu.sync_copy(idx_hbm.at[pl.ds(b0, CB)], idx_v)
            gcopies = [
                pltpu.async_copy(
                    table_hbm.at[idx_v.at[j]],
                    rows_v.at[pl.ds(j * L, L)],
                    gsem,
                )
                for j in range(CB)
            ]
            for cp in gcopies:
                cp.wait()

            def elem_body(e, carry2):
                r0 = e * L
                a0 = rows_v[r0, 0:HALF] + rows_v[r0 + 1, 0:HALF]
                b0_ = rows_v[r0, HALF:D] + rows_v[r0 + 1, HALF:D]
                a1 = rows_v[r0 + 2, 0:HALF] + rows_v[r0 + 3, 0:HALF]
                b1 = rows_v[r0 + 2, HALF:D] + rows_v[r0 + 3, HALF:D]
                for l in range(4, L, 2):
                    a0 = a0 + rows_v[r0 + l, 0:HALF]
                    b0_ = b0_ + rows_v[r0 + l, HALF:D]
                    a1 = a1 + rows_v[r0 + l + 1, 0:HALF]
                    b1 = b1 + rows_v[r0 + l + 1, HALF:D]
                o0 = e * D
                out_v[pl.ds(o0, HALF)] = (a0 + a1) * (1.0 / L)
                out_v[pl.ds(o0 + HALF, HALF)] = (b0_ + b1) * (1.0 / L)
                return carry2

            lax.fori_loop(0, CB, elem_body, 0)
            pltpu.sync_copy(out_v, out_hbm.at[pl.ds(b0 * D, CB * D)])
            return carry

        lax.fori_loop(0, NCH, chunk_body, 0)

    return bow(batch, table_lin)


def kernel(batch, table):
    tlin = _detranspose(table.T)
    table_lin = tlin.reshape(VPAD, D)
    return _bow(batch, table_lin).reshape(B, D)


# speedup vs baseline: 2.1813x; 1.9310x over previous
import functools

import jax
import jax.numpy as jnp
from jax import lax
from jax.experimental import pallas as pl
from jax.experimental.pallas import tpu as pltpu
from jax.experimental.pallas import tpu_sc as plsc

B = 16384
L = 50
D = 32
NW = 32
EPW = B // NW
CB = 32
NCH = EPW // CB
ROWS = CB * L
HALF = D // 2

V = 1000000
BLK = 128
NFULL = V // BLK
TAIL = V - NFULL * BLK
NBLK = NFULL + 1
VPAD = NBLK * BLK
KMAX = (NBLK + NW - 1) // NW


def _detranspose(table_t):
    mesh = plsc.VectorSubcoreMesh(core_axis_name="c", subcore_axis_name="s")

    @functools.partial(
        pl.kernel,
        mesh=mesh,
        out_type=jax.ShapeDtypeStruct((VPAD * D,), jnp.float32),
        scratch_types=[
            pltpu.VMEM((8, BLK), jnp.float32),
            pltpu.VMEM((8, BLK), jnp.float32),
            pltpu.VMEM((8, BLK), jnp.float32),
            pltpu.VMEM((8, BLK), jnp.float32),
            pltpu.VMEM((8, BLK), jnp.float32),
            pltpu.VMEM((8, BLK), jnp.float32),
            pltpu.VMEM((8, BLK), jnp.float32),
            pltpu.VMEM((8, BLK), jnp.float32),
            pltpu.VMEM((BLK * D,), jnp.float32),
            pltpu.VMEM((BLK * D,), jnp.float32),
            pltpu.SemaphoreType.DMA,
            pltpu.SemaphoreType.DMA,
            pltpu.SemaphoreType.DMA,
            pltpu.SemaphoreType.DMA,
        ],
        compiler_params=pltpu.CompilerParams(
            use_tc_tiling_on_sc=True, needs_layout_passes=False
        ),
    )
    def detr(t_hbm, o_hbm, b00, b01, b02, b03, b10, b11, b12, b13,
             st0, st1, gs0, gs1, os0, os1):
        wid = lax.axis_index("s") * 2 + lax.axis_index("c")
        blks = ((b00, b01, b02, b03), (b10, b11, b12, b13))
        sts = (st0, st1)
        gsems = (gs0, gs1)
        osems = (os0, os1)

        io = lax.iota(jnp.int32, 16)
        c8 = io & 7
        hlf = io >> 3
        sbase = [hlf * D + g * 8 + c8 for g in range(4)]

        def fire(k, p):
            tc = wid + k * NW

            @pl.when(tc < NFULL)
            def _():
                for g in range(4):
                    pltpu.async_copy(
                        t_hbm.at[pl.ds(g * 8, 8), pl.ds(tc * BLK, BLK)],
                        blks[p][g],
                        gsems[p],
                    )

            @pl.when(tc == NFULL)
            def _():
                for g in range(4):
                    pltpu.async_copy(
                        t_hbm.at[pl.ds(g * 8, 8), pl.ds(tc * BLK, TAIL)],
                        blks[p][g].at[:, pl.ds(0, TAIL)],
                        gsems[p],
                    )

        def transpose_rows(p, nrow):
            @plsc.parallel_loop(0, nrow // 2, 1, unroll=8)
            def _(i2):
                i = i2 * 2
                col = hlf + i
                off = i * D
                for g in range(4):
                    vals = plsc.load_gather(blks[p][g], [c8, col])
                    plsc.store_scatter(sts[p], [sbase[g] + off], vals)

        def compute(k, p):
            tc = wid + k * NW

            @pl.when(tc <= NFULL)
            def _():
                pltpu.make_async_copy(
                    sts[p], o_hbm.at[pl.ds(0, BLK * D)], osems[p]
                ).wait()

            @pl.when(tc < NFULL)
            def _():
                for g in range(4):
                    pltpu.make_async_copy(
                        t_hbm.at[pl.ds(0, 8), pl.ds(0, BLK)],
                        blks[p][g],
                        gsems[p],
                    ).wait()
                transpose_rows(p, BLK)

            @pl.when(tc == NFULL)
            def _():
                for g in range(4):
                    pltpu.make_async_copy(
                        t_hbm.at[pl.ds(0, 8), pl.ds(0, TAIL)],
                        blks[p][g].at[:, pl.ds(0, TAIL)],
                        gsems[p],
                    ).wait()
                transpose_rows(p, TAIL)

            @pl.when(tc <= NFULL)
            def _():
                pltpu.async_copy(
                    sts[p],
                    o_hbm.at[pl.ds(tc * (BLK * D), BLK * D)],
                    osems[p],
                )

        for p in range(2):
            pltpu.async_copy(
                sts[p],
                o_hbm.at[pl.ds((wid + p * NW) * (BLK * D), BLK * D)],
                osems[p],
            )

        fire(0, 0)

        def loop_body(k2, carry):
            for ph in range(2):
                k = k2 * 2 + ph
                p = ph
                fire(k + 1, 1 - p)
                compute(k, p)
            return carry

        lax.fori_loop(0, KMAX // 2, loop_body, 0)
        compute(KMAX - 1, 0)

        for p in range(2):
            pltpu.make_async_copy(
                sts[p], o_hbm.at[pl.ds(0, BLK * D)], osems[p]
            ).wait()

    return detr(table_t)


def _bow(batch, table_lin):
    mesh = plsc.VectorSubcoreMesh(core_axis_name="c", subcore_axis_name="s")

    @functools.partial(
        pl.kernel,
        mesh=mesh,
        out_type=jax.ShapeDtypeStruct((B * D,), jnp.float32),
        scratch_types=[
            pltpu.VMEM((CB, L), jnp.int32),
            pltpu.VMEM((ROWS, D), jnp.float32),
            pltpu.VMEM((CB * D,), jnp.float32),
            pltpu.SemaphoreType.DMA,
        ],
        compiler_params=pltpu.CompilerParams(use_tc_tiling_on_sc=False),
    )
    def bow(idx_hbm, table_hbm, out_hbm, idx_v, rows_v, out_v, gsem):
        wid = lax.axis_index("s") * 2 + lax.axis_index("c")

        def chunk_body(c, carry):
            b0 = wid * EPW + c * CB
            pltpu.sync_copy(idx_hbm.at[pl.ds(b0, CB)], idx_v)
            gcopies = [
                pltpu.async_copy(
                    table_hbm.at[idx_v.at[j]],
                    rows_v.at[pl.ds(j * L, L)],
                    gsem,
                )
                for j in range(CB)
            ]
            for cp in gcopies:
                cp.wait()

            def elem_body(e, carry2):
                r0 = e * L
                a0 = rows_v[r0, 0:HALF] + rows_v[r0 + 1, 0:HALF]
                b0_ = rows_v[r0, HALF:D] + rows_v[r0 + 1, HALF:D]
                a1 = rows_v[r0 + 2, 0:HALF] + rows_v[r0 + 3, 0:HALF]
                b1 = rows_v[r0 + 2, HALF:D] + rows_v[r0 + 3, HALF:D]
                for l in range(4, L, 2):
                    a0 = a0 + rows_v[r0 + l, 0:HALF]
                    b0_ = b0_ + rows_v[r0 + l, HALF:D]
                    a1 = a1 + rows_v[r0 + l + 1, 0:HALF]
                    b1 = b1 + rows_v[r0 + l + 1, HALF:D]
                o0 = e * D
                out_v[pl.ds(o0, HALF)] = (a0 + a1) * (1.0 / L)
                out_v[pl.ds(o0 + HALF, HALF)] = (b0_ + b1) * (1.0 / L)
                return carry2

            lax.fori_loop(0, CB, elem_body, 0)
            pltpu.sync_copy(out_v, out_hbm.at[pl.ds(b0 * D, CB * D)])
            return carry

        lax.fori_loop(0, NCH, chunk_body, 0)

    return bow(batch, table_lin)


def kernel(batch, table):
    tlin = _detranspose(table.T)
    table_lin = tlin.reshape(VPAD, D)
    return _bow(batch, table_lin).reshape(B, D)


# speedup vs baseline: 2.4220x; 1.1104x over previous
import functools

import jax
import jax.numpy as jnp
from jax import lax
from jax.experimental import pallas as pl
from jax.experimental.pallas import tpu as pltpu
from jax.experimental.pallas import tpu_sc as plsc

B = 16384
L = 50
D = 32
NW = 32
EPW = B // NW
CB = 32
NCH = EPW // CB
ROWS = CB * L
HALF = D // 2

V = 1000000
BLK = 128
NFULL = V // BLK
TAIL = V - NFULL * BLK
NBLK = NFULL + 1
VPAD = NBLK * BLK
KMAX = (NBLK + NW - 1) // NW


def _detranspose(table_t):
    mesh = plsc.VectorSubcoreMesh(core_axis_name="c", subcore_axis_name="s")

    @functools.partial(
        pl.kernel,
        mesh=mesh,
        out_type=jax.ShapeDtypeStruct((VPAD * D,), jnp.float32),
        scratch_types=[
            pltpu.VMEM((8, BLK), jnp.float32),
            pltpu.VMEM((8, BLK), jnp.float32),
            pltpu.VMEM((8, BLK), jnp.float32),
            pltpu.VMEM((8, BLK), jnp.float32),
            pltpu.VMEM((8, BLK), jnp.float32),
            pltpu.VMEM((8, BLK), jnp.float32),
            pltpu.VMEM((8, BLK), jnp.float32),
            pltpu.VMEM((8, BLK), jnp.float32),
            pltpu.VMEM((BLK * D,), jnp.float32),
            pltpu.VMEM((BLK * D,), jnp.float32),
            pltpu.SemaphoreType.DMA,
            pltpu.SemaphoreType.DMA,
            pltpu.SemaphoreType.DMA,
            pltpu.SemaphoreType.DMA,
        ],
        compiler_params=pltpu.CompilerParams(
            use_tc_tiling_on_sc=True, needs_layout_passes=False
        ),
    )
    def detr(t_hbm, o_hbm, b00, b01, b02, b03, b10, b11, b12, b13,
             st0, st1, gs0, gs1, os0, os1):
        wid = lax.axis_index("s") * 2 + lax.axis_index("c")
        blks = ((b00, b01, b02, b03), (b10, b11, b12, b13))
        sts = (st0, st1)
        gsems = (gs0, gs1)
        osems = (os0, os1)

        io = lax.iota(jnp.int32, 16)
        c8 = io & 7
        hlf = io >> 3
        sbase = [hlf * D + g * 8 + c8 for g in range(4)]

        def fire(k, p):
            tc = wid + k * NW

            @pl.when(tc < NFULL)
            def _():
                for g in range(4):
                    pltpu.async_copy(
                        t_hbm.at[pl.ds(g * 8, 8), pl.ds(tc * BLK, BLK)],
                        blks[p][g],
                        gsems[p],
                    )

            @pl.when(tc == NFULL)
            def _():
                for g in range(4):
                    pltpu.async_copy(
                        t_hbm.at[pl.ds(g * 8, 8), pl.ds(tc * BLK, TAIL)],
                        blks[p][g].at[:, pl.ds(0, TAIL)],
                        gsems[p],
                    )

        def transpose_rows(p, nrow):
            @plsc.parallel_loop(0, nrow // 2, 1, unroll=8)
            def _(i2):
                i = i2 * 2
                col = hlf + i
                off = i * D
                for g in range(4):
                    vals = plsc.load_gather(blks[p][g], [c8, col])
                    plsc.store_scatter(sts[p], [sbase[g] + off], vals)

        def compute(k, p):
            tc = wid + k * NW

            @pl.when(tc <= NFULL)
            def _():
                pltpu.make_async_copy(
                    sts[p], o_hbm.at[pl.ds(0, BLK * D)], osems[p]
                ).wait()

            @pl.when(tc < NFULL)
            def _():
                for g in range(4):
                    pltpu.make_async_copy(
                        t_hbm.at[pl.ds(0, 8), pl.ds(0, BLK)],
                        blks[p][g],
                        gsems[p],
                    ).wait()
                transpose_rows(p, BLK)

            @pl.when(tc == NFULL)
            def _():
                for g in range(4):
                    pltpu.make_async_copy(
                        t_hbm.at[pl.ds(0, 8), pl.ds(0, TAIL)],
                        blks[p][g].at[:, pl.ds(0, TAIL)],
                        gsems[p],
                    ).wait()
                transpose_rows(p, TAIL)

            @pl.when(tc <= NFULL)
            def _():
                pltpu.async_copy(
                    sts[p],
                    o_hbm.at[pl.ds(tc * (BLK * D), BLK * D)],
                    osems[p],
                )

        for p in range(2):
            pltpu.async_copy(
                sts[p],
                o_hbm.at[pl.ds((wid + p * NW) * (BLK * D), BLK * D)],
                osems[p],
            )

        fire(0, 0)

        def loop_body(k2, carry):
            for ph in range(2):
                k = k2 * 2 + ph
                p = ph
                fire(k + 1, 1 - p)
                compute(k, p)
            return carry

        lax.fori_loop(0, KMAX // 2, loop_body, 0)
        compute(KMAX - 1, 0)

        for p in range(2):
            pltpu.make_async_copy(
                sts[p], o_hbm.at[pl.ds(0, BLK * D)], osems[p]
            ).wait()

    return detr(table_t)


def _bow(batch, table_lin):
    mesh = plsc.VectorSubcoreMesh(core_axis_name="c", subcore_axis_name="s")

    @functools.partial(
        pl.kernel,
        mesh=mesh,
        out_type=jax.ShapeDtypeStruct((B * D,), jnp.float32),
        scratch_types=[
            pltpu.VMEM((CB, L), jnp.int32),
            pltpu.VMEM((CB, L), jnp.int32),
            pltpu.VMEM((ROWS, D), jnp.float32),
            pltpu.VMEM((ROWS, D), jnp.float32),
            pltpu.VMEM((CB * D,), jnp.float32),
            pltpu.SemaphoreType.DMA,
            pltpu.SemaphoreType.DMA,
        ],
        compiler_params=pltpu.CompilerParams(use_tc_tiling_on_sc=False),
    )
    def bow(idx_hbm, table_hbm, out_hbm, idx0, idx1, rows0, rows1, out_v,
            gs0, gs1):
        wid = lax.axis_index("s") * 2 + lax.axis_index("c")
        idxs = (idx0, idx1)
        rows = (rows0, rows1)
        gsems = (gs0, gs1)

        def fire(c, p):
            @pl.when(c < NCH)
            def _():
                b0 = wid * EPW + c * CB
                pltpu.sync_copy(idx_hbm.at[pl.ds(b0, CB)], idxs[p])
                for j in range(CB):
                    pltpu.async_copy(
                        table_hbm.at[idxs[p].at[j]],
                        rows[p].at[pl.ds(j * L, L)],
                        gsems[p],
                    )

        def reduce_chunk(c, p):
            rows_v = rows[p]
            pltpu.make_async_copy(
                table_hbm.at[pl.ds(0, ROWS)], rows_v, gsems[p]
            ).wait()
            fire(c + 1, 1 - p)

            def elem_body(e, carry2):
                r0 = e * L
                a0 = rows_v[r0, 0:HALF] + rows_v[r0 + 1, 0:HALF]
                b0_ = rows_v[r0, HALF:D] + rows_v[r0 + 1, HALF:D]
                a1 = rows_v[r0 + 2, 0:HALF] + rows_v[r0 + 3, 0:HALF]
                b1 = rows_v[r0 + 2, HALF:D] + rows_v[r0 + 3, HALF:D]
                for l in range(4, L, 2):
                    a0 = a0 + rows_v[r0 + l, 0:HALF]
                    b0_ = b0_ + rows_v[r0 + l, HALF:D]
                    a1 = a1 + rows_v[r0 + l + 1, 0:HALF]
                    b1 = b1 + rows_v[r0 + l + 1, HALF:D]
                o0 = e * D
                out_v[pl.ds(o0, HALF)] = (a0 + a1) * (1.0 / L)
                out_v[pl.ds(o0 + HALF, HALF)] = (b0_ + b1) * (1.0 / L)
                return carry2

            lax.fori_loop(0, CB, elem_body, 0)
            b0 = wid * EPW + c * CB
            pltpu.sync_copy(out_v, out_hbm.at[pl.ds(b0 * D, CB * D)])

        fire(0, 0)

        def chunk_pair(c2, carry):
            reduce_chunk(c2 * 2, 0)
            reduce_chunk(c2 * 2 + 1, 1)
            return carry

        lax.fori_loop(0, NCH // 2, chunk_pair, 0)

    return bow(batch, table_lin)


def kernel(batch, table):
    tlin = _detranspose(table.T)
    table_lin = tlin.reshape(VPAD, D)
    return _bow(batch, table_lin).reshape(B, D)
